# baseline (speedup 1.0000x reference)
import jax
import jax.numpy as jnp
from jax.experimental import pallas as pl

N = 8192
BR = 256


def _bt_block(r_rows_ref, e_cols_ref, out_ref):
    den = e_cols_ref[...] * r_rows_ref[...] + 1.0
    bits = jax.lax.bitcast_convert_type(den, jnp.int32)
    r = jax.lax.bitcast_convert_type(jnp.int32(0x7EF127EA) - bits, jnp.float32)
    r = r * (2.0 - den * r)
    r = r * (2.0 - den * r)
    out_ref[...] = r


def kernel(ability):
    r_rows = jnp.exp(-ability).reshape(N, 1)
    e_cols = jnp.exp(ability).reshape(1, N)
    return pl.pallas_call(
        _bt_block,
        grid=(N // BR,),
        in_specs=[
            pl.BlockSpec((BR, 1), lambda i: (i, 0)),
            pl.BlockSpec((1, N), lambda i: (0, 0)),
        ],
        out_specs=pl.BlockSpec((BR, N), lambda i: (i, 0)),
        out_shape=jax.ShapeDtypeStruct((N, N), jnp.float32),
    )(r_rows, e_cols)


# speedup vs baseline: 1.8410x; 1.8410x over previous
import jax
import jax.numpy as jnp
from jax.experimental import pallas as pl

N = 8192
BR = 512


def _bt_block(r_rows_ref, e_cols_ref, out_ref):
    den = e_cols_ref[...] * r_rows_ref[...] + 1.0
    out_ref[...] = 1.0 / den


def kernel(ability):
    r_rows = jnp.exp(-ability).reshape(N, 1)
    e_cols = jnp.exp(ability).reshape(1, N)
    return pl.pallas_call(
        _bt_block,
        grid=(N // BR,),
        in_specs=[
            pl.BlockSpec((BR, 1), lambda i: (i, 0)),
            pl.BlockSpec((1, N), lambda i: (0, 0)),
        ],
        out_specs=pl.BlockSpec((BR, N), lambda i: (i, 0)),
        out_shape=jax.ShapeDtypeStruct((N, N), jnp.float32),
    )(r_rows, e_cols)


# speedup vs baseline: 1.8426x; 1.0009x over previous
import jax
import jax.numpy as jnp
from jax.experimental import pallas as pl
from jax.experimental.pallas import tpu as pltpu

N = 8192
BR = 1024
BC = 4096


def _bt_block(r_rows_ref, e_cols_ref, out_ref):
    den = e_cols_ref[...] * r_rows_ref[...] + 1.0
    out_ref[...] = 1.0 / den


def kernel(ability):
    r_rows = jnp.exp(-ability).reshape(N, 1)
    e_cols = jnp.exp(ability).reshape(1, N)
    return pl.pallas_call(
        _bt_block,
        grid=(N // BR, N // BC),
        in_specs=[
            pl.BlockSpec((BR, 1), lambda i, j: (i, 0)),
            pl.BlockSpec((1, BC), lambda i, j: (0, j)),
        ],
        out_specs=pl.BlockSpec((BR, BC), lambda i, j: (i, j)),
        out_shape=jax.ShapeDtypeStruct((N, N), jnp.float32),
    )(r_rows, e_cols)
